# baseline (speedup 1.0000x reference)
import functools

import jax
import jax.numpy as jnp
from jax.experimental import pallas as pl
from jax.experimental.pallas import tpu as pltpu

_VMEM_LIMIT = 48 * 1024 * 1024


def _scse_kernel(x00_ref, x01_ref, x10_ref, x11_ref, w1t_ref, b1_ref,
                 w2t_ref, b2_ref, ws_ref, bs_ref, o_ref, *, hw2, inv_hw):
    for i, (a_ref, b_ref) in enumerate(((x00_ref, x01_ref),
                                        (x10_ref, x11_ref))):
        xa = a_ref[0]
        xb = b_ref[0]

        pooled = (jnp.sum(xa, axis=0, keepdims=True)
                  + jnp.sum(xb, axis=0, keepdims=True)) * inv_hw
        z = jnp.dot(pooled, w1t_ref[...],
                    preferred_element_type=jnp.float32) + b1_ref[...]
        z = jnp.maximum(z, 0.0)
        s = jnp.dot(z, w2t_ref[...],
                    preferred_element_type=jnp.float32) + b2_ref[...]
        cse = jax.nn.sigmoid(s)

        for j, xh in enumerate((xa, xb)):
            sp = jnp.dot(xh, ws_ref[...],
                         preferred_element_type=jnp.float32) + bs_ref[0]
            sse = jax.nn.sigmoid(sp)
            o_ref[i, j * hw2:(j + 1) * hw2, :] = xh * (cse + sse)


def kernel(x, w1, b1, w2, b2, ws, bs):
    N, C, H, W = x.shape
    HW = H * W
    HW2 = HW // 2
    mid = w1.shape[0]
    B = 2

    xt = jnp.transpose(x, (0, 2, 3, 1)).reshape(N, HW, C)
    xh = xt.reshape(2 * N, HW2, C)

    w1t = w1.astype(jnp.float32).T
    w2t = w2.astype(jnp.float32).T
    b1r = b1.reshape(1, mid).astype(jnp.float32)
    b2r = b2.reshape(1, C).astype(jnp.float32)
    ws_col = ws.reshape(1, C).T.astype(jnp.float32)
    bs_smem = bs.reshape(1).astype(jnp.float32)

    out = pl.pallas_call(
        functools.partial(_scse_kernel, hw2=HW2, inv_hw=1.0 / HW),
        out_shape=jax.ShapeDtypeStruct((N, HW, C), jnp.float32),
        grid_spec=pltpu.PrefetchScalarGridSpec(
            num_scalar_prefetch=0,
            grid=(N // B,),
            in_specs=[
                pl.BlockSpec((1, HW2, C), lambda n: (4 * n, 0, 0)),
                pl.BlockSpec((1, HW2, C), lambda n: (4 * n + 1, 0, 0)),
                pl.BlockSpec((1, HW2, C), lambda n: (4 * n + 2, 0, 0)),
                pl.BlockSpec((1, HW2, C), lambda n: (4 * n + 3, 0, 0)),
                pl.BlockSpec((C, mid), lambda n: (0, 0)),
                pl.BlockSpec((1, mid), lambda n: (0, 0)),
                pl.BlockSpec((mid, C), lambda n: (0, 0)),
                pl.BlockSpec((1, C), lambda n: (0, 0)),
                pl.BlockSpec((C, 1), lambda n: (0, 0)),
                pl.BlockSpec(memory_space=pltpu.MemorySpace.SMEM),
            ],
            out_specs=pl.BlockSpec((B, HW, C), lambda n: (n, 0, 0)),
        ),
        compiler_params=pltpu.CompilerParams(
            dimension_semantics=("parallel",),
            vmem_limit_bytes=_VMEM_LIMIT),
    )(xh, xh, xh, xh, w1t, b1r, w2t, b2r, ws_col, bs_smem)

    return jnp.transpose(out.reshape(N, H, W, C), (0, 3, 1, 2))
